# baseline (speedup 1.0000x reference)
import functools

import jax
import jax.numpy as jnp
from jax import lax
from jax.experimental import pallas as pl
from jax.experimental.pallas import tpu as pltpu
from jax.experimental.pallas import tpu_sc as plsc


def _tc_body(alpha_ref, vf_ref, vg_ref, bone_ref, wv_ref, bv_ref, wb_ref,
             bb_ref, out_ref, bproj_s):
    @pl.when(pl.program_id(0) == 0)
    def _():
        bproj_s[:, :] = (
            jnp.dot(bone_ref[:, :], wb_ref[:, :],
                    preferred_element_type=jnp.float32)
            + bb_ref[:, :]
        )

    vproj = (
        jnp.dot(vf_ref[:, :], wv_ref[:, :], preferred_element_type=jnp.float32)
        + bv_ref[:, :]
    )
    scores = jax.lax.dot_general(
        vproj, bproj_s[:, :],
        dimension_numbers=(((1,), (1,)), ((), ())),
        preferred_element_type=jnp.float32,
    )
    alpha = alpha_ref[0, 0]
    out_ref[:, :] = scores * jnp.exp(-alpha * vg_ref[:, :])


def _biased_scores(vertex_features, bone_embeddings, vol_geo, W_v, b_v, W_b,
                   b_b, alpha, interpret=False):
    n, vfd = vertex_features.shape
    b, bfd = bone_embeddings.shape
    cd = W_v.shape[1]
    blk = 2000
    return pl.pallas_call(
        _tc_body,
        grid=(n // blk,),
        in_specs=[
            pl.BlockSpec(memory_space=pltpu.SMEM),
            pl.BlockSpec((blk, vfd), lambda i: (i, 0)),
            pl.BlockSpec((blk, b), lambda i: (i, 0)),
            pl.BlockSpec((b, bfd), lambda i: (0, 0)),
            pl.BlockSpec((vfd, cd), lambda i: (0, 0)),
            pl.BlockSpec((1, cd), lambda i: (0, 0)),
            pl.BlockSpec((bfd, cd), lambda i: (0, 0)),
            pl.BlockSpec((1, cd), lambda i: (0, 0)),
        ],
        out_specs=pl.BlockSpec((blk, b), lambda i: (i, 0)),
        out_shape=jax.ShapeDtypeStruct((n, b), jnp.float32),
        scratch_shapes=[pltpu.VMEM((b, cd), jnp.float32)],
        compiler_params=pltpu.CompilerParams(
            dimension_semantics=("arbitrary",),
        ),
        interpret=interpret,
    )(
        jnp.reshape(alpha.astype(jnp.float32), (1, 1)),
        vertex_features,
        vol_geo,
        bone_embeddings,
        W_v,
        jnp.reshape(b_v, (1, cd)),
        W_b,
        jnp.reshape(b_b, (1, cd)),
    )


_L = 16
_NW = 32
_UNROLL = 8


_NB = 256


def _sc_topk_softmax_body(biased_hbm, out_hbm, buf, ebuf):
    n_groups = biased_hbm.shape[0] // (_L * _NB)
    per = (n_groups + _NW - 1) // _NW
    w = lax.axis_index("s") * 2 + lax.axis_index("c")
    start = w * per
    n_g = jnp.maximum(jnp.minimum(per, n_groups - start), 0)

    row_base = lax.iota(jnp.int32, _L) * _NB
    neg_inf = jnp.full((_L,), -jnp.inf, dtype=jnp.float32)

    def group_body(g, carry):
        base = (start + g) * (_L * _NB)
        pltpu.sync_copy(biased_hbm.at[pl.ds(base, _L * _NB)], buf)

        def p1(i, ts):
            t1, t2, t3, t4 = ts
            for j in range(_UNROLL):
                idx = row_base + (i * _UNROLL + j)
                v = plsc.load_gather(buf, [idx])
                lo1 = jnp.minimum(t1, v)
                t1 = jnp.maximum(t1, v)
                lo2 = jnp.minimum(t2, lo1)
                t2 = jnp.maximum(t2, lo1)
                lo3 = jnp.minimum(t3, lo2)
                t3 = jnp.maximum(t3, lo2)
                t4 = jnp.maximum(t4, lo3)
            return (t1, t2, t3, t4)

        t1, t2, t3, t4 = lax.fori_loop(
            0, _NB // _UNROLL, p1, (neg_inf, neg_inf, neg_inf, neg_inf))

        def p2(i, acc):
            for j in range(_UNROLL):
                idx = row_base + (i * _UNROLL + j)
                v = plsc.load_gather(buf, [idx])
                e = jnp.where(v >= t4, jnp.exp(v - t1), 0.0)
                plsc.store_scatter(ebuf, [idx], e)
                acc = acc + e
            return acc

        denom = lax.fori_loop(0, _NB // _UNROLL, p2,
                              jnp.zeros((_L,), jnp.float32))
        recip = 1.0 / denom

        def p3(i, c2):
            for j in range(_UNROLL):
                idx = row_base + (i * _UNROLL + j)
                e = plsc.load_gather(ebuf, [idx])
                plsc.store_scatter(ebuf, [idx], e * recip)
            return c2

        lax.fori_loop(0, _NB // _UNROLL, p3, 0)
        pltpu.sync_copy(ebuf, out_hbm.at[pl.ds(base, _L * _NB)])
        return carry

    lax.fori_loop(0, n_g, group_body, 0)


def _sc_topk_softmax(biased):
    n, nb = biased.shape
    mesh = plsc.VectorSubcoreMesh(core_axis_name="c", subcore_axis_name="s")
    flat = pl.kernel(
        _sc_topk_softmax_body,
        out_type=jax.ShapeDtypeStruct((n * nb,), jnp.float32),
        mesh=mesh,
        scratch_types=[
            pltpu.VMEM((_L * _NB,), jnp.float32),
            pltpu.VMEM((_L * _NB,), jnp.float32),
        ],
        compiler_params=pltpu.CompilerParams(needs_layout_passes=False),
    )(jnp.reshape(biased, (n * nb,)))
    return jnp.reshape(flat, (n, nb))


@functools.partial(jax.jit, static_argnames=("interpret",))
def kernel(vertex_features, bone_embeddings, vol_geo, W_v, b_v, W_b, b_b,
           alpha, interpret=False):
    biased = _biased_scores(vertex_features, bone_embeddings, vol_geo, W_v,
                            b_v, W_b, b_b, alpha, interpret=interpret)
    return _sc_topk_softmax(biased)


# speedup vs baseline: 1.3383x; 1.3383x over previous
import functools

import jax
import jax.numpy as jnp
from jax import lax
from jax.experimental import pallas as pl
from jax.experimental.pallas import tpu as pltpu
from jax.experimental.pallas import tpu_sc as plsc


def _tc_body(alpha_ref, vf_ref, vg_ref, bone_ref, wv_ref, bv_ref, wb_ref,
             bb_ref, out_ref, bproj_s):
    @pl.when(pl.program_id(0) == 0)
    def _():
        bproj_s[:, :] = (
            jnp.dot(bone_ref[:, :], wb_ref[:, :],
                    preferred_element_type=jnp.float32)
            + bb_ref[:, :]
        )

    vproj = (
        jnp.dot(vf_ref[:, :], wv_ref[:, :], preferred_element_type=jnp.float32)
        + bv_ref[:, :]
    )
    scores = jax.lax.dot_general(
        vproj, bproj_s[:, :],
        dimension_numbers=(((1,), (1,)), ((), ())),
        preferred_element_type=jnp.float32,
    )
    alpha = alpha_ref[0, 0]
    out_ref[:, :] = scores * jnp.exp(-alpha * vg_ref[:, :])


def _biased_scores(vertex_features, bone_embeddings, vol_geo, W_v, b_v, W_b,
                   b_b, alpha, interpret=False):
    n, vfd = vertex_features.shape
    b, bfd = bone_embeddings.shape
    cd = W_v.shape[1]
    blk = 2000
    return pl.pallas_call(
        _tc_body,
        grid=(n // blk,),
        in_specs=[
            pl.BlockSpec(memory_space=pltpu.SMEM),
            pl.BlockSpec((blk, vfd), lambda i: (i, 0)),
            pl.BlockSpec((blk, b), lambda i: (i, 0)),
            pl.BlockSpec((b, bfd), lambda i: (0, 0)),
            pl.BlockSpec((vfd, cd), lambda i: (0, 0)),
            pl.BlockSpec((1, cd), lambda i: (0, 0)),
            pl.BlockSpec((bfd, cd), lambda i: (0, 0)),
            pl.BlockSpec((1, cd), lambda i: (0, 0)),
        ],
        out_specs=pl.BlockSpec((blk, b), lambda i: (i, 0)),
        out_shape=jax.ShapeDtypeStruct((n, b), jnp.float32),
        scratch_shapes=[pltpu.VMEM((b, cd), jnp.float32)],
        compiler_params=pltpu.CompilerParams(
            dimension_semantics=("arbitrary",),
        ),
        interpret=interpret,
    )(
        jnp.reshape(alpha.astype(jnp.float32), (1, 1)),
        vertex_features,
        vol_geo,
        bone_embeddings,
        W_v,
        jnp.reshape(b_v, (1, cd)),
        W_b,
        jnp.reshape(b_b, (1, cd)),
    )


_L = 16
_NW = 32
_UNROLL = 8


_NB = 256
_CHUNK_ROWS = 80
_GROUPS_PER_CHUNK = _CHUNK_ROWS // _L
_CHAINS = 8
_COLS_PER_CHAIN = _NB // _CHAINS


def _merge4(a, b):
    x1 = jnp.maximum(a[0], b[3])
    x2 = jnp.maximum(a[1], b[2])
    x3 = jnp.maximum(a[2], b[1])
    x4 = jnp.maximum(a[3], b[0])
    y1 = jnp.maximum(x1, x3)
    y3 = jnp.minimum(x1, x3)
    y2 = jnp.maximum(x2, x4)
    y4 = jnp.minimum(x2, x4)
    return (jnp.maximum(y1, y2), jnp.minimum(y1, y2),
            jnp.maximum(y3, y4), jnp.minimum(y3, y4))


def _sc_topk_softmax_body(biased_hbm, out_hbm, buf, ebuf):
    n_chunks = biased_hbm.shape[0] // (_CHUNK_ROWS * _NB)
    w = lax.axis_index("s") * 2 + lax.axis_index("c")
    n_i = (n_chunks - w + _NW - 1) // _NW

    lane_base = lax.iota(jnp.int32, _L) * _NB
    neg_inf = jnp.full((_L,), -jnp.inf, dtype=jnp.float32)

    def chunk_body(i, carry):
        base = (w + i * _NW) * (_CHUNK_ROWS * _NB)
        pltpu.sync_copy(biased_hbm.at[pl.ds(base, _CHUNK_ROWS * _NB)], buf)

        for g in range(_GROUPS_PER_CHUNK):
            gbase = lane_base + g * (_L * _NB)

            def p1(ci, ts):
                out = []
                for k in range(_CHAINS):
                    t1, t2, t3, t4 = ts[4 * k:4 * k + 4]
                    idx = gbase + (k * _COLS_PER_CHAIN + ci)
                    v = plsc.load_gather(buf, [idx])
                    lo1 = jnp.minimum(t1, v)
                    t1 = jnp.maximum(t1, v)
                    lo2 = jnp.minimum(t2, lo1)
                    t2 = jnp.maximum(t2, lo1)
                    lo3 = jnp.minimum(t3, lo2)
                    t3 = jnp.maximum(t3, lo2)
                    t4 = jnp.maximum(t4, lo3)
                    out.extend((t1, t2, t3, t4))
                return tuple(out)

            ts = lax.fori_loop(0, _COLS_PER_CHAIN, p1,
                               (neg_inf,) * (4 * _CHAINS))

            stacks = [tuple(ts[4 * k:4 * k + 4]) for k in range(_CHAINS)]
            while len(stacks) > 1:
                stacks = [_merge4(stacks[2 * j], stacks[2 * j + 1])
                          for j in range(len(stacks) // 2)]
            s1, s2, s3, s4 = stacks[0]

            denom = (1.0 + jnp.exp(s2 - s1) + jnp.exp(s3 - s1)
                     + jnp.exp(s4 - s1))
            recip = 1.0 / denom

            def p2(ci, c2):
                for k in range(_CHAINS):
                    idx = gbase + (k * _COLS_PER_CHAIN + ci)
                    v = plsc.load_gather(buf, [idx])
                    e = jnp.where(v >= s4, jnp.exp(v - s1) * recip, 0.0)
                    plsc.store_scatter(ebuf, [idx], e)
                return c2

            lax.fori_loop(0, _COLS_PER_CHAIN, p2, 0)

        pltpu.sync_copy(ebuf, out_hbm.at[pl.ds(base, _CHUNK_ROWS * _NB)])
        return carry

    lax.fori_loop(0, n_i, chunk_body, 0)


def _sc_topk_softmax(biased):
    n, nb = biased.shape
    mesh = plsc.VectorSubcoreMesh(core_axis_name="c", subcore_axis_name="s")
    flat = pl.kernel(
        _sc_topk_softmax_body,
        out_type=jax.ShapeDtypeStruct((n * nb,), jnp.float32),
        mesh=mesh,
        scratch_types=[
            pltpu.VMEM((_CHUNK_ROWS * _NB,), jnp.float32),
            pltpu.VMEM((_CHUNK_ROWS * _NB,), jnp.float32),
        ],
        compiler_params=pltpu.CompilerParams(needs_layout_passes=False),
    )(jnp.reshape(biased, (n * nb,)))
    return jnp.reshape(flat, (n, nb))


@functools.partial(jax.jit, static_argnames=("interpret",))
def kernel(vertex_features, bone_embeddings, vol_geo, W_v, b_v, W_b, b_b,
           alpha, interpret=False):
    biased = _biased_scores(vertex_features, bone_embeddings, vol_geo, W_v,
                            b_v, W_b, b_b, alpha, interpret=interpret)
    return _sc_topk_softmax(biased)


# speedup vs baseline: 1.6647x; 1.2439x over previous
import functools

import jax
import jax.numpy as jnp
from jax import lax
from jax.experimental import pallas as pl
from jax.experimental.pallas import tpu as pltpu
from jax.experimental.pallas import tpu_sc as plsc


def _tc_body(alpha_ref, vf_ref, vg_ref, bone_ref, wv_ref, bv_ref, wb_ref,
             bb_ref, out_ref, bproj_s):
    @pl.when(pl.program_id(0) == 0)
    def _():
        bproj_s[:, :] = (
            jnp.dot(bone_ref[:, :], wb_ref[:, :],
                    preferred_element_type=jnp.float32)
            + bb_ref[:, :]
        )

    vproj = (
        jnp.dot(vf_ref[:, :], wv_ref[:, :], preferred_element_type=jnp.float32)
        + bv_ref[:, :]
    )
    scores = jax.lax.dot_general(
        vproj, bproj_s[:, :],
        dimension_numbers=(((1,), (1,)), ((), ())),
        preferred_element_type=jnp.float32,
    )
    alpha = alpha_ref[0, 0]
    out_ref[:, :] = scores * jnp.exp(-alpha * vg_ref[:, :])


def _biased_scores(vertex_features, bone_embeddings, vol_geo, W_v, b_v, W_b,
                   b_b, alpha, interpret=False):
    n, vfd = vertex_features.shape
    b, bfd = bone_embeddings.shape
    cd = W_v.shape[1]
    blk = 2000
    return pl.pallas_call(
        _tc_body,
        grid=(n // blk,),
        in_specs=[
            pl.BlockSpec(memory_space=pltpu.SMEM),
            pl.BlockSpec((blk, vfd), lambda i: (i, 0)),
            pl.BlockSpec((blk, b), lambda i: (i, 0)),
            pl.BlockSpec((b, bfd), lambda i: (0, 0)),
            pl.BlockSpec((vfd, cd), lambda i: (0, 0)),
            pl.BlockSpec((1, cd), lambda i: (0, 0)),
            pl.BlockSpec((bfd, cd), lambda i: (0, 0)),
            pl.BlockSpec((1, cd), lambda i: (0, 0)),
        ],
        out_specs=pl.BlockSpec((blk, b), lambda i: (i, 0)),
        out_shape=jax.ShapeDtypeStruct((n, b), jnp.float32),
        scratch_shapes=[pltpu.VMEM((b, cd), jnp.float32)],
        compiler_params=pltpu.CompilerParams(
            dimension_semantics=("arbitrary",),
        ),
        interpret=interpret,
    )(
        jnp.reshape(alpha.astype(jnp.float32), (1, 1)),
        vertex_features,
        vol_geo,
        bone_embeddings,
        W_v,
        jnp.reshape(b_v, (1, cd)),
        W_b,
        jnp.reshape(b_b, (1, cd)),
    )


_L = 16
_NW = 32
_UNROLL = 8


_NB = 256
_CHUNK_ROWS = 80
_GROUPS_PER_CHUNK = _CHUNK_ROWS // _L
_CHAINS = 8
_COLS_PER_CHAIN = _NB // _CHAINS


def _merge4(a, b):
    x1 = jnp.maximum(a[0], b[3])
    x2 = jnp.maximum(a[1], b[2])
    x3 = jnp.maximum(a[2], b[1])
    x4 = jnp.maximum(a[3], b[0])
    y1 = jnp.maximum(x1, x3)
    y3 = jnp.minimum(x1, x3)
    y2 = jnp.maximum(x2, x4)
    y4 = jnp.minimum(x2, x4)
    return (jnp.maximum(y1, y2), jnp.minimum(y1, y2),
            jnp.maximum(y3, y4), jnp.minimum(y3, y4))


def _sc_topk_softmax_body(biased_hbm, out_hbm, buf, ebuf):
    n_chunks = biased_hbm.shape[0] // (_CHUNK_ROWS * _NB)
    w = lax.axis_index("s") * 2 + lax.axis_index("c")
    n_i = (n_chunks - w + _NW - 1) // _NW

    lane_base = lax.iota(jnp.int32, _L) * _NB
    neg_inf = jnp.full((_L,), -jnp.inf, dtype=jnp.float32)

    def chunk_body(i, carry):
        base = (w + i * _NW) * (_CHUNK_ROWS * _NB)
        pltpu.sync_copy(biased_hbm.at[pl.ds(base, _CHUNK_ROWS * _NB)], buf)

        for g in range(_GROUPS_PER_CHUNK):
            gbase = lane_base + g * (_L * _NB)

            def p1(ci, ts):
                vs = [plsc.load_gather(buf, [gbase + (k * _COLS_PER_CHAIN + ci)])
                      for k in range(_CHAINS)]
                t1s = [ts[4 * k] for k in range(_CHAINS)]
                t2s = [ts[4 * k + 1] for k in range(_CHAINS)]
                t3s = [ts[4 * k + 2] for k in range(_CHAINS)]
                t4s = [ts[4 * k + 3] for k in range(_CHAINS)]
                lo1 = [jnp.minimum(t1s[k], vs[k]) for k in range(_CHAINS)]
                t1s = [jnp.maximum(t1s[k], vs[k]) for k in range(_CHAINS)]
                lo2 = [jnp.minimum(t2s[k], lo1[k]) for k in range(_CHAINS)]
                t2s = [jnp.maximum(t2s[k], lo1[k]) for k in range(_CHAINS)]
                lo3 = [jnp.minimum(t3s[k], lo2[k]) for k in range(_CHAINS)]
                t3s = [jnp.maximum(t3s[k], lo2[k]) for k in range(_CHAINS)]
                t4s = [jnp.maximum(t4s[k], lo3[k]) for k in range(_CHAINS)]
                out = []
                for k in range(_CHAINS):
                    out.extend((t1s[k], t2s[k], t3s[k], t4s[k]))
                return tuple(out)

            ts = lax.fori_loop(0, _COLS_PER_CHAIN, p1,
                               (neg_inf,) * (4 * _CHAINS))

            stacks = [tuple(ts[4 * k:4 * k + 4]) for k in range(_CHAINS)]
            while len(stacks) > 1:
                stacks = [_merge4(stacks[2 * j], stacks[2 * j + 1])
                          for j in range(len(stacks) // 2)]
            s1, s2, s3, s4 = stacks[0]

            denom = (1.0 + jnp.exp(s2 - s1) + jnp.exp(s3 - s1)
                     + jnp.exp(s4 - s1))
            recip = 1.0 / denom

            def p2(ci, c2):
                idxs = [gbase + (k * _COLS_PER_CHAIN + ci)
                        for k in range(_CHAINS)]
                vs = [plsc.load_gather(buf, [idxs[k]])
                      for k in range(_CHAINS)]
                exps = [jnp.exp(vs[k] - s1) for k in range(_CHAINS)]
                es = [jnp.where(vs[k] >= s4, exps[k] * recip, 0.0)
                      for k in range(_CHAINS)]
                for k in range(_CHAINS):
                    plsc.store_scatter(ebuf, [idxs[k]], es[k])
                return c2

            lax.fori_loop(0, _COLS_PER_CHAIN, p2, 0)

        pltpu.sync_copy(ebuf, out_hbm.at[pl.ds(base, _CHUNK_ROWS * _NB)])
        return carry

    lax.fori_loop(0, n_i, chunk_body, 0)


def _sc_topk_softmax(biased):
    n, nb = biased.shape
    mesh = plsc.VectorSubcoreMesh(core_axis_name="c", subcore_axis_name="s")
    flat = pl.kernel(
        _sc_topk_softmax_body,
        out_type=jax.ShapeDtypeStruct((n * nb,), jnp.float32),
        mesh=mesh,
        scratch_types=[
            pltpu.VMEM((_CHUNK_ROWS * _NB,), jnp.float32),
            pltpu.VMEM((_CHUNK_ROWS * _NB,), jnp.float32),
        ],
        compiler_params=pltpu.CompilerParams(needs_layout_passes=False),
    )(jnp.reshape(biased, (n * nb,)))
    return jnp.reshape(flat, (n, nb))


@functools.partial(jax.jit, static_argnames=("interpret",))
def kernel(vertex_features, bone_embeddings, vol_geo, W_v, b_v, W_b, b_b,
           alpha, interpret=False):
    biased = _biased_scores(vertex_features, bone_embeddings, vol_geo, W_v,
                            b_v, W_b, b_b, alpha, interpret=interpret)
    return _sc_topk_softmax(biased)


# speedup vs baseline: 4.6497x; 2.7931x over previous
import functools

import jax
import jax.numpy as jnp
from jax import lax
from jax.experimental import pallas as pl
from jax.experimental.pallas import tpu as pltpu
from jax.experimental.pallas import tpu_sc as plsc


def _tc_body(alpha_ref, vf_ref, vg_ref, bone_ref, wv_ref, bv_ref, wb_ref,
             bb_ref, out_ref, bproj_s):
    @pl.when(pl.program_id(0) == 0)
    def _():
        bproj_s[:, :] = (
            jnp.dot(bone_ref[:, :], wb_ref[:, :],
                    preferred_element_type=jnp.float32)
            + bb_ref[:, :]
        )

    vproj = (
        jnp.dot(vf_ref[:, :], wv_ref[:, :], preferred_element_type=jnp.float32)
        + bv_ref[:, :]
    )
    scores = jax.lax.dot_general(
        vproj, bproj_s[:, :],
        dimension_numbers=(((1,), (1,)), ((), ())),
        preferred_element_type=jnp.float32,
    )
    alpha = alpha_ref[0, 0]
    out_ref[:, :] = scores * jnp.exp(-alpha * vg_ref[:, :])


def _biased_scores(vertex_features, bone_embeddings, vol_geo, W_v, b_v, W_b,
                   b_b, alpha, interpret=False):
    n, vfd = vertex_features.shape
    b, bfd = bone_embeddings.shape
    cd = W_v.shape[1]
    blk = 2000
    return pl.pallas_call(
        _tc_body,
        grid=(n // blk,),
        in_specs=[
            pl.BlockSpec(memory_space=pltpu.SMEM),
            pl.BlockSpec((blk, vfd), lambda i: (i, 0)),
            pl.BlockSpec((blk, b), lambda i: (i, 0)),
            pl.BlockSpec((b, bfd), lambda i: (0, 0)),
            pl.BlockSpec((vfd, cd), lambda i: (0, 0)),
            pl.BlockSpec((1, cd), lambda i: (0, 0)),
            pl.BlockSpec((bfd, cd), lambda i: (0, 0)),
            pl.BlockSpec((1, cd), lambda i: (0, 0)),
        ],
        out_specs=pl.BlockSpec((blk, b), lambda i: (i, 0)),
        out_shape=jax.ShapeDtypeStruct((n, b), jnp.float32),
        scratch_shapes=[pltpu.VMEM((b, cd), jnp.float32)],
        compiler_params=pltpu.CompilerParams(
            dimension_semantics=("arbitrary",),
        ),
        interpret=interpret,
    )(
        jnp.reshape(alpha.astype(jnp.float32), (1, 1)),
        vertex_features,
        vol_geo,
        bone_embeddings,
        W_v,
        jnp.reshape(b_v, (1, cd)),
        W_b,
        jnp.reshape(b_b, (1, cd)),
    )


_L = 16
_NW = 32
_UNROLL = 8


_NB = 256
_CHUNK_ROWS = 80
_GROUPS_PER_CHUNK = _CHUNK_ROWS // _L
_CHAINS = 8
_COLS_PER_CHAIN = _NB // _CHAINS


def _merge4(a, b):
    x1 = jnp.maximum(a[0], b[3])
    x2 = jnp.maximum(a[1], b[2])
    x3 = jnp.maximum(a[2], b[1])
    x4 = jnp.maximum(a[3], b[0])
    y1 = jnp.maximum(x1, x3)
    y3 = jnp.minimum(x1, x3)
    y2 = jnp.maximum(x2, x4)
    y4 = jnp.minimum(x2, x4)
    return (jnp.maximum(y1, y2), jnp.minimum(y1, y2),
            jnp.maximum(y3, y4), jnp.minimum(y3, y4))


def _sc_topk_softmax_body(biased_hbm, out_hbm, buf, ebuf):
    n_chunks = biased_hbm.shape[0] // (_CHUNK_ROWS * _NB)
    w = lax.axis_index("s") * 2 + lax.axis_index("c")
    n_i = (n_chunks - w + _NW - 1) // _NW

    lane_iota = lax.iota(jnp.int32, _L)
    lane_base = lane_iota * _NB
    neg_inf = jnp.full((_L,), -jnp.inf, dtype=jnp.float32)

    def chunk_body(i, carry):
        base = (w + i * _NW) * (_CHUNK_ROWS * _NB)
        pltpu.sync_copy(biased_hbm.at[pl.ds(base, _CHUNK_ROWS * _NB)], buf)

        for g in range(_GROUPS_PER_CHUNK):
            gbase = lane_base + g * (_L * _NB)

            def p1(ci, ts):
                idxs = [gbase
                        + ((lane_iota + (k * _COLS_PER_CHAIN + ci)) & (_NB - 1))
                        for k in range(_CHAINS)]
                vs = [plsc.load_gather(buf, [idxs[k]])
                      for k in range(_CHAINS)]
                t1s = [ts[4 * k] for k in range(_CHAINS)]
                t2s = [ts[4 * k + 1] for k in range(_CHAINS)]
                t3s = [ts[4 * k + 2] for k in range(_CHAINS)]
                t4s = [ts[4 * k + 3] for k in range(_CHAINS)]
                lo1 = [jnp.minimum(t1s[k], vs[k]) for k in range(_CHAINS)]
                t1s = [jnp.maximum(t1s[k], vs[k]) for k in range(_CHAINS)]
                lo2 = [jnp.minimum(t2s[k], lo1[k]) for k in range(_CHAINS)]
                t2s = [jnp.maximum(t2s[k], lo1[k]) for k in range(_CHAINS)]
                lo3 = [jnp.minimum(t3s[k], lo2[k]) for k in range(_CHAINS)]
                t3s = [jnp.maximum(t3s[k], lo2[k]) for k in range(_CHAINS)]
                t4s = [jnp.maximum(t4s[k], lo3[k]) for k in range(_CHAINS)]
                out = []
                for k in range(_CHAINS):
                    out.extend((t1s[k], t2s[k], t3s[k], t4s[k]))
                return tuple(out)

            ts = lax.fori_loop(0, _COLS_PER_CHAIN, p1,
                               (neg_inf,) * (4 * _CHAINS))

            stacks = [tuple(ts[4 * k:4 * k + 4]) for k in range(_CHAINS)]
            while len(stacks) > 1:
                stacks = [_merge4(stacks[2 * j], stacks[2 * j + 1])
                          for j in range(len(stacks) // 2)]
            s1, s2, s3, s4 = stacks[0]

            denom = (1.0 + jnp.exp(s2 - s1) + jnp.exp(s3 - s1)
                     + jnp.exp(s4 - s1))
            recip = 1.0 / denom

            def p2(ci, c2):
                idxs = [gbase
                        + ((lane_iota + (k * _COLS_PER_CHAIN + ci)) & (_NB - 1))
                        for k in range(_CHAINS)]
                vs = [plsc.load_gather(buf, [idxs[k]])
                      for k in range(_CHAINS)]
                exps = [jnp.exp(vs[k] - s1) for k in range(_CHAINS)]
                es = [jnp.where(vs[k] >= s4, exps[k] * recip, 0.0)
                      for k in range(_CHAINS)]
                for k in range(_CHAINS):
                    plsc.store_scatter(ebuf, [idxs[k]], es[k])
                return c2

            lax.fori_loop(0, _COLS_PER_CHAIN, p2, 0)

        pltpu.sync_copy(ebuf, out_hbm.at[pl.ds(base, _CHUNK_ROWS * _NB)])
        return carry

    lax.fori_loop(0, n_i, chunk_body, 0)


def _sc_topk_softmax(biased):
    n, nb = biased.shape
    mesh = plsc.VectorSubcoreMesh(core_axis_name="c", subcore_axis_name="s")
    flat = pl.kernel(
        _sc_topk_softmax_body,
        out_type=jax.ShapeDtypeStruct((n * nb,), jnp.float32),
        mesh=mesh,
        scratch_types=[
            pltpu.VMEM((_CHUNK_ROWS * _NB,), jnp.float32),
            pltpu.VMEM((_CHUNK_ROWS * _NB,), jnp.float32),
        ],
        compiler_params=pltpu.CompilerParams(needs_layout_passes=False),
    )(jnp.reshape(biased, (n * nb,)))
    return jnp.reshape(flat, (n, nb))


@functools.partial(jax.jit, static_argnames=("interpret",))
def kernel(vertex_features, bone_embeddings, vol_geo, W_v, b_v, W_b, b_b,
           alpha, interpret=False):
    biased = _biased_scores(vertex_features, bone_embeddings, vol_geo, W_v,
                            b_v, W_b, b_b, alpha, interpret=interpret)
    return _sc_topk_softmax(biased)


# speedup vs baseline: 5.4840x; 1.1794x over previous
import functools

import jax
import jax.numpy as jnp
from jax import lax
from jax.experimental import pallas as pl
from jax.experimental.pallas import tpu as pltpu
from jax.experimental.pallas import tpu_sc as plsc


def _tc_body(alpha_ref, vf_ref, vg_ref, bone_ref, wv_ref, bv_ref, wb_ref,
             bb_ref, out_ref, bproj_s):
    @pl.when(pl.program_id(0) == 0)
    def _():
        bproj_s[:, :] = (
            jnp.dot(bone_ref[:, :], wb_ref[:, :],
                    preferred_element_type=jnp.float32)
            + bb_ref[:, :]
        )

    vproj = (
        jnp.dot(vf_ref[:, :], wv_ref[:, :], preferred_element_type=jnp.float32)
        + bv_ref[:, :]
    )
    scores = jax.lax.dot_general(
        vproj, bproj_s[:, :],
        dimension_numbers=(((1,), (1,)), ((), ())),
        preferred_element_type=jnp.float32,
    )
    alpha = alpha_ref[0, 0]
    out_ref[:, :] = scores * jnp.exp(-alpha * vg_ref[:, :])


def _biased_scores(vertex_features, bone_embeddings, vol_geo, W_v, b_v, W_b,
                   b_b, alpha, interpret=False):
    n, vfd = vertex_features.shape
    b, bfd = bone_embeddings.shape
    cd = W_v.shape[1]
    blk = 2000
    return pl.pallas_call(
        _tc_body,
        grid=(n // blk,),
        in_specs=[
            pl.BlockSpec(memory_space=pltpu.SMEM),
            pl.BlockSpec((blk, vfd), lambda i: (i, 0)),
            pl.BlockSpec((blk, b), lambda i: (i, 0)),
            pl.BlockSpec((b, bfd), lambda i: (0, 0)),
            pl.BlockSpec((vfd, cd), lambda i: (0, 0)),
            pl.BlockSpec((1, cd), lambda i: (0, 0)),
            pl.BlockSpec((bfd, cd), lambda i: (0, 0)),
            pl.BlockSpec((1, cd), lambda i: (0, 0)),
        ],
        out_specs=pl.BlockSpec((blk, b), lambda i: (i, 0)),
        out_shape=jax.ShapeDtypeStruct((n, b), jnp.float32),
        scratch_shapes=[pltpu.VMEM((b, cd), jnp.float32)],
        compiler_params=pltpu.CompilerParams(
            dimension_semantics=("arbitrary",),
        ),
        interpret=interpret,
    )(
        jnp.reshape(alpha.astype(jnp.float32), (1, 1)),
        vertex_features,
        vol_geo,
        bone_embeddings,
        W_v,
        jnp.reshape(b_v, (1, cd)),
        W_b,
        jnp.reshape(b_b, (1, cd)),
    )


_L = 16
_NW = 32
_UNROLL = 8


_NB = 256
_CHUNK_ROWS = 80
_GROUPS_PER_CHUNK = _CHUNK_ROWS // _L
_CHAINS = 8
_COLS_PER_CHAIN = _NB // _CHAINS


def _merge4(a, b):
    x1 = jnp.maximum(a[0], b[3])
    x2 = jnp.maximum(a[1], b[2])
    x3 = jnp.maximum(a[2], b[1])
    x4 = jnp.maximum(a[3], b[0])
    y1 = jnp.maximum(x1, x3)
    y3 = jnp.minimum(x1, x3)
    y2 = jnp.maximum(x2, x4)
    y4 = jnp.minimum(x2, x4)
    return (jnp.maximum(y1, y2), jnp.minimum(y1, y2),
            jnp.maximum(y3, y4), jnp.minimum(y3, y4))


def _sc_topk_softmax_body(biased_hbm, out_hbm, buf0, buf1, ebuf0, ebuf1,
                          isem0, isem1, osem0, osem1):
    n_chunks = biased_hbm.shape[0] // (_CHUNK_ROWS * _NB)
    w = lax.axis_index("s") * 2 + lax.axis_index("c")
    n_i = (n_chunks - w + _NW - 1) // _NW

    bufs = (buf0, buf1)
    ebufs = (ebuf0, ebuf1)
    isems = (isem0, isem1)
    osems = (osem0, osem1)
    ch = _CHUNK_ROWS * _NB

    def src(i):
        return biased_hbm.at[pl.ds((w + i * _NW) * ch, ch)]

    def dst(i):
        return out_hbm.at[pl.ds((w + i * _NW) * ch, ch)]

    lane_iota = lax.iota(jnp.int32, _L)
    lane_base = lane_iota * _NB
    neg_inf = jnp.full((_L,), -jnp.inf, dtype=jnp.float32)

    def compute_chunk(buf, ebuf):
        for g in range(_GROUPS_PER_CHUNK):
            gbase = lane_base + g * (_L * _NB)

            def p1(ci, ts):
                idxs = [gbase
                        + ((lane_iota + (k * _COLS_PER_CHAIN + ci)) & (_NB - 1))
                        for k in range(_CHAINS)]
                vs = [plsc.load_gather(buf, [idxs[k]])
                      for k in range(_CHAINS)]
                t1s = [ts[4 * k] for k in range(_CHAINS)]
                t2s = [ts[4 * k + 1] for k in range(_CHAINS)]
                t3s = [ts[4 * k + 2] for k in range(_CHAINS)]
                t4s = [ts[4 * k + 3] for k in range(_CHAINS)]
                lo1 = [jnp.minimum(t1s[k], vs[k]) for k in range(_CHAINS)]
                t1s = [jnp.maximum(t1s[k], vs[k]) for k in range(_CHAINS)]
                lo2 = [jnp.minimum(t2s[k], lo1[k]) for k in range(_CHAINS)]
                t2s = [jnp.maximum(t2s[k], lo1[k]) for k in range(_CHAINS)]
                lo3 = [jnp.minimum(t3s[k], lo2[k]) for k in range(_CHAINS)]
                t3s = [jnp.maximum(t3s[k], lo2[k]) for k in range(_CHAINS)]
                t4s = [jnp.maximum(t4s[k], lo3[k]) for k in range(_CHAINS)]
                out = []
                for k in range(_CHAINS):
                    out.extend((t1s[k], t2s[k], t3s[k], t4s[k]))
                return tuple(out)

            ts = lax.fori_loop(0, _COLS_PER_CHAIN, p1,
                               (neg_inf,) * (4 * _CHAINS))

            stacks = [tuple(ts[4 * k:4 * k + 4]) for k in range(_CHAINS)]
            while len(stacks) > 1:
                stacks = [_merge4(stacks[2 * j], stacks[2 * j + 1])
                          for j in range(len(stacks) // 2)]
            s1, s2, s3, s4 = stacks[0]

            denom = (1.0 + jnp.exp(s2 - s1) + jnp.exp(s3 - s1)
                     + jnp.exp(s4 - s1))
            recip = 1.0 / denom

            def p2(ci, c2):
                idxs = [gbase
                        + ((lane_iota + (k * _COLS_PER_CHAIN + ci)) & (_NB - 1))
                        for k in range(_CHAINS)]
                vs = [plsc.load_gather(buf, [idxs[k]])
                      for k in range(_CHAINS)]
                exps = [jnp.exp(vs[k] - s1) for k in range(_CHAINS)]
                es = [jnp.where(vs[k] >= s4, exps[k] * recip, 0.0)
                      for k in range(_CHAINS)]
                for k in range(_CHAINS):
                    plsc.store_scatter(ebuf, [idxs[k]], es[k])
                return c2

            lax.fori_loop(0, _COLS_PER_CHAIN, p2, 0)

    pltpu.make_async_copy(src(0), bufs[0], isems[0]).start()

    def pair_body(p, carry):
        for b in (0, 1):
            i = 2 * p + b

            @pl.when(i < n_i)
            def _():
                pltpu.make_async_copy(src(i), bufs[b], isems[b]).wait()

                @pl.when(i + 1 < n_i)
                def _():
                    pltpu.make_async_copy(src(i + 1), bufs[1 - b],
                                          isems[1 - b]).start()

                @pl.when(i >= 2)
                def _():
                    pltpu.make_async_copy(ebufs[b], dst(i), osems[b]).wait()

                compute_chunk(bufs[b], ebufs[b])
                pltpu.make_async_copy(ebufs[b], dst(i), osems[b]).start()
        return carry

    lax.fori_loop(0, (n_i + 1) // 2, pair_body, 0)
    pltpu.make_async_copy(ebufs[0], dst(0), osems[0]).wait()
    pltpu.make_async_copy(ebufs[1], dst(0), osems[1]).wait()


def _sc_topk_softmax(biased):
    n, nb = biased.shape
    mesh = plsc.VectorSubcoreMesh(core_axis_name="c", subcore_axis_name="s")
    flat = pl.kernel(
        _sc_topk_softmax_body,
        out_type=jax.ShapeDtypeStruct((n * nb,), jnp.float32),
        mesh=mesh,
        scratch_types=[
            pltpu.VMEM((_CHUNK_ROWS * _NB,), jnp.float32),
            pltpu.VMEM((_CHUNK_ROWS * _NB,), jnp.float32),
            pltpu.VMEM((_CHUNK_ROWS * _NB,), jnp.float32),
            pltpu.VMEM((_CHUNK_ROWS * _NB,), jnp.float32),
            pltpu.SemaphoreType.DMA,
            pltpu.SemaphoreType.DMA,
            pltpu.SemaphoreType.DMA,
            pltpu.SemaphoreType.DMA,
        ],
        compiler_params=pltpu.CompilerParams(needs_layout_passes=False),
    )(jnp.reshape(biased, (n * nb,)))
    return jnp.reshape(flat, (n, nb))


@functools.partial(jax.jit, static_argnames=("interpret",))
def kernel(vertex_features, bone_embeddings, vol_geo, W_v, b_v, W_b, b_b,
           alpha, interpret=False):
    biased = _biased_scores(vertex_features, bone_embeddings, vol_geo, W_v,
                            b_v, W_b, b_b, alpha, interpret=interpret)
    return _sc_topk_softmax(biased)


# speedup vs baseline: 8.6339x; 1.5744x over previous
import functools

import jax
import jax.numpy as jnp
from jax import lax
from jax.experimental import pallas as pl
from jax.experimental.pallas import tpu as pltpu
from jax.experimental.pallas import tpu_sc as plsc


def _tc_body(alpha_ref, vf_ref, vg_ref, bone_ref, wv_ref, bv_ref, wb_ref,
             bb_ref, out_ref, bproj_s):
    @pl.when(pl.program_id(0) == 0)
    def _():
        bproj_s[:, :] = (
            jnp.dot(bone_ref[:, :], wb_ref[:, :],
                    preferred_element_type=jnp.float32)
            + bb_ref[:, :]
        )

    vproj = (
        jnp.dot(vf_ref[:, :], wv_ref[:, :], preferred_element_type=jnp.float32)
        + bv_ref[:, :]
    )
    scores = jax.lax.dot_general(
        vproj, bproj_s[:, :],
        dimension_numbers=(((1,), (1,)), ((), ())),
        preferred_element_type=jnp.float32,
    )
    alpha = alpha_ref[0, 0]
    out_ref[:, :] = scores * jnp.exp(-alpha * vg_ref[:, :])


def _biased_scores(vertex_features, bone_embeddings, vol_geo, W_v, b_v, W_b,
                   b_b, alpha, interpret=False):
    n, vfd = vertex_features.shape
    b, bfd = bone_embeddings.shape
    cd = W_v.shape[1]
    blk = 2000
    return pl.pallas_call(
        _tc_body,
        grid=(n // blk,),
        in_specs=[
            pl.BlockSpec(memory_space=pltpu.SMEM),
            pl.BlockSpec((blk, vfd), lambda i: (i, 0)),
            pl.BlockSpec((blk, b), lambda i: (i, 0)),
            pl.BlockSpec((b, bfd), lambda i: (0, 0)),
            pl.BlockSpec((vfd, cd), lambda i: (0, 0)),
            pl.BlockSpec((1, cd), lambda i: (0, 0)),
            pl.BlockSpec((bfd, cd), lambda i: (0, 0)),
            pl.BlockSpec((1, cd), lambda i: (0, 0)),
        ],
        out_specs=pl.BlockSpec((blk, b), lambda i: (i, 0)),
        out_shape=jax.ShapeDtypeStruct((n, b), jnp.float32),
        scratch_shapes=[pltpu.VMEM((b, cd), jnp.float32)],
        compiler_params=pltpu.CompilerParams(
            dimension_semantics=("arbitrary",),
        ),
        interpret=interpret,
    )(
        jnp.reshape(alpha.astype(jnp.float32), (1, 1)),
        vertex_features,
        vol_geo,
        bone_embeddings,
        W_v,
        jnp.reshape(b_v, (1, cd)),
        W_b,
        jnp.reshape(b_b, (1, cd)),
    )


_L = 16
_NW = 32
_UNROLL = 8


_NB = 256
_CHUNK_ROWS = 80
_GROUPS_PER_CHUNK = _CHUNK_ROWS // _L
_CHAINS = 8
_COLS_PER_CHAIN = _NB // _CHAINS


def _merge4(a, b):
    x1 = jnp.maximum(a[0], b[3])
    x2 = jnp.maximum(a[1], b[2])
    x3 = jnp.maximum(a[2], b[1])
    x4 = jnp.maximum(a[3], b[0])
    y1 = jnp.maximum(x1, x3)
    y3 = jnp.minimum(x1, x3)
    y2 = jnp.maximum(x2, x4)
    y4 = jnp.minimum(x2, x4)
    return (jnp.maximum(y1, y2), jnp.minimum(y1, y2),
            jnp.maximum(y3, y4), jnp.minimum(y3, y4))


def _sc_topk_softmax_body(biased_hbm, out_hbm, buf0, buf1, ebuf0, ebuf1,
                          isem0, isem1, osem0, osem1):
    n_chunks = biased_hbm.shape[0] // _CHUNK_ROWS
    w = lax.axis_index("s") * 2 + lax.axis_index("c")
    n_i = (n_chunks - w + _NW - 1) // _NW

    bufs = (buf0, buf1)
    ebufs = (ebuf0, ebuf1)
    isems = (isem0, isem1)
    osems = (osem0, osem1)

    def src(i):
        return biased_hbm.at[pl.ds((w + i * _NW) * _CHUNK_ROWS, _CHUNK_ROWS)]

    def dst(i):
        return out_hbm.at[pl.ds((w + i * _NW) * _CHUNK_ROWS, _CHUNK_ROWS)]

    lane_iota = lax.iota(jnp.int32, _L)
    neg_inf = jnp.full((_L,), -jnp.inf, dtype=jnp.float32)

    def compute_chunk(buf, ebuf):
        for g in range(_GROUPS_PER_CHUNK):
            rows = lane_iota + g * _L

            def p1(ci, ts):
                idxs = [(lane_iota + (k * _COLS_PER_CHAIN + ci)) & (_NB - 1)
                        for k in range(_CHAINS)]
                vs = [plsc.load_gather(buf, [rows, idxs[k]])
                      for k in range(_CHAINS)]
                t1s = [ts[4 * k] for k in range(_CHAINS)]
                t2s = [ts[4 * k + 1] for k in range(_CHAINS)]
                t3s = [ts[4 * k + 2] for k in range(_CHAINS)]
                t4s = [ts[4 * k + 3] for k in range(_CHAINS)]
                lo1 = [jnp.minimum(t1s[k], vs[k]) for k in range(_CHAINS)]
                t1s = [jnp.maximum(t1s[k], vs[k]) for k in range(_CHAINS)]
                lo2 = [jnp.minimum(t2s[k], lo1[k]) for k in range(_CHAINS)]
                t2s = [jnp.maximum(t2s[k], lo1[k]) for k in range(_CHAINS)]
                lo3 = [jnp.minimum(t3s[k], lo2[k]) for k in range(_CHAINS)]
                t3s = [jnp.maximum(t3s[k], lo2[k]) for k in range(_CHAINS)]
                t4s = [jnp.maximum(t4s[k], lo3[k]) for k in range(_CHAINS)]
                out = []
                for k in range(_CHAINS):
                    out.extend((t1s[k], t2s[k], t3s[k], t4s[k]))
                return tuple(out)

            ts = lax.fori_loop(0, _COLS_PER_CHAIN, p1,
                               (neg_inf,) * (4 * _CHAINS))

            stacks = [tuple(ts[4 * k:4 * k + 4]) for k in range(_CHAINS)]
            while len(stacks) > 1:
                stacks = [_merge4(stacks[2 * j], stacks[2 * j + 1])
                          for j in range(len(stacks) // 2)]
            s1, s2, s3, s4 = stacks[0]

            denom = (1.0 + jnp.exp(s2 - s1) + jnp.exp(s3 - s1)
                     + jnp.exp(s4 - s1))
            recip = 1.0 / denom

            def p2(ci, c2):
                idxs = [(lane_iota + (k * _COLS_PER_CHAIN + ci)) & (_NB - 1)
                        for k in range(_CHAINS)]
                vs = [plsc.load_gather(buf, [rows, idxs[k]])
                      for k in range(_CHAINS)]
                exps = [jnp.exp(vs[k] - s1) for k in range(_CHAINS)]
                es = [jnp.where(vs[k] >= s4, exps[k] * recip, 0.0)
                      for k in range(_CHAINS)]
                for k in range(_CHAINS):
                    plsc.store_scatter(ebuf, [rows, idxs[k]], es[k])
                return c2

            lax.fori_loop(0, _COLS_PER_CHAIN, p2, 0)

    pltpu.make_async_copy(src(0), bufs[0], isems[0]).start()

    def pair_body(p, carry):
        for b in (0, 1):
            i = 2 * p + b

            @pl.when(i < n_i)
            def _():
                pltpu.make_async_copy(src(i), bufs[b], isems[b]).wait()

                @pl.when(i + 1 < n_i)
                def _():
                    pltpu.make_async_copy(src(i + 1), bufs[1 - b],
                                          isems[1 - b]).start()

                @pl.when(i >= 2)
                def _():
                    pltpu.make_async_copy(ebufs[b], dst(i), osems[b]).wait()

                compute_chunk(bufs[b], ebufs[b])
                pltpu.make_async_copy(ebufs[b], dst(i), osems[b]).start()
        return carry

    lax.fori_loop(0, (n_i + 1) // 2, pair_body, 0)
    pltpu.make_async_copy(ebufs[0], dst(0), osems[0]).wait()
    pltpu.make_async_copy(ebufs[1], dst(0), osems[1]).wait()


def _sc_topk_softmax(biased):
    n, nb = biased.shape
    mesh = plsc.VectorSubcoreMesh(core_axis_name="c", subcore_axis_name="s")
    return pl.kernel(
        _sc_topk_softmax_body,
        out_type=jax.ShapeDtypeStruct((n, nb), jnp.float32),
        mesh=mesh,
        scratch_types=[
            pltpu.VMEM((_CHUNK_ROWS, _NB), jnp.float32),
            pltpu.VMEM((_CHUNK_ROWS, _NB), jnp.float32),
            pltpu.VMEM((_CHUNK_ROWS, _NB), jnp.float32),
            pltpu.VMEM((_CHUNK_ROWS, _NB), jnp.float32),
            pltpu.SemaphoreType.DMA,
            pltpu.SemaphoreType.DMA,
            pltpu.SemaphoreType.DMA,
            pltpu.SemaphoreType.DMA,
        ],
        compiler_params=pltpu.CompilerParams(
            needs_layout_passes=False,
            use_tc_tiling_on_sc=True,
        ),
    )(biased)


@functools.partial(jax.jit, static_argnames=("interpret",))
def kernel(vertex_features, bone_embeddings, vol_geo, W_v, b_v, W_b, b_b,
           alpha, interpret=False):
    biased = _biased_scores(vertex_features, bone_embeddings, vol_geo, W_v,
                            b_v, W_b, b_b, alpha, interpret=interpret)
    return _sc_topk_softmax(biased)
